# Initial kernel scaffold; baseline (speedup 1.0000x reference)
import jax
import jax.numpy as jnp
from jax.experimental import pallas as pl


def kernel(input_ids, wte):
    raise NotImplementedError("write your pallas kernel here")


# speedup vs baseline: 1.4429x; 1.4429x over previous
import functools

import jax
import jax.numpy as jnp
from jax import lax
from jax.experimental import pallas as pl
from jax.experimental.pallas import tpu as pltpu
from jax.experimental.pallas import tpu_sc as plsc

_VOCAB = 50304
_N_EMBD = 2048
_BATCH = 4
_SEQ = 2048


@functools.lru_cache(maxsize=None)
def _make_gather(B: int, D: int):
    info = plsc.get_sparse_core_info()
    NC, NS = info.num_cores, info.num_subcores
    NW = NC * NS
    b_per_w = B // NW
    CH = 16
    n_chunks = b_per_w // CH
    mesh = plsc.VectorSubcoreMesh(core_axis_name="c", subcore_axis_name="s")

    @functools.partial(
        pl.kernel,
        mesh=mesh,
        out_type=jax.ShapeDtypeStruct((B, D), jnp.float32),
        scratch_types=[
            pltpu.VMEM((b_per_w,), jnp.int32),
            pltpu.VMEM((CH, D), jnp.float32),
            pltpu.SemaphoreType.DMA,
        ],
    )
    def gather_kernel(idx_hbm, table_hbm, out_hbm, idx_v, rows_v, sem):
        wid = lax.axis_index("s") * NC + lax.axis_index("c")
        base = wid * b_per_w
        pltpu.sync_copy(idx_hbm.at[pl.ds(base, b_per_w)], idx_v)

        def body(c, carry):
            pltpu.async_copy(
                table_hbm.at[idx_v.at[pl.ds(c * CH, CH)]], rows_v, sem
            ).wait()
            pltpu.sync_copy(rows_v, out_hbm.at[pl.ds(base + c * CH, CH)])
            return carry

        lax.fori_loop(0, n_chunks, body, 0)

    return gather_kernel


def kernel(input_ids, wte):
    input_shape = input_ids.shape
    flat_ids = input_ids.reshape(-1).astype(jnp.int32)
    B = flat_ids.shape[0]
    D = wte.shape[1]
    out = _make_gather(B, D)(flat_ids, wte)
    return out.reshape(input_shape[0], input_shape[1], D)


# speedup vs baseline: 1.6575x; 1.1487x over previous
import functools

import jax
import jax.numpy as jnp
from jax import lax
from jax.experimental import pallas as pl
from jax.experimental.pallas import tpu as pltpu
from jax.experimental.pallas import tpu_sc as plsc

_NBUF = 4
_CH = 8


@functools.lru_cache(maxsize=None)
def _make_gather(B: int, D: int):
    info = plsc.get_sparse_core_info()
    NC, NS = info.num_cores, info.num_subcores
    NW = NC * NS
    b_per_w = B // NW
    n_chunks = b_per_w // _CH
    n_rings = n_chunks // _NBUF
    assert n_chunks % _NBUF == 0 and n_rings >= 3
    mesh = plsc.VectorSubcoreMesh(core_axis_name="c", subcore_axis_name="s")

    @functools.partial(
        pl.kernel,
        mesh=mesh,
        out_type=jax.ShapeDtypeStruct((B, D), jnp.float32),
        scratch_types=[
            pltpu.VMEM((b_per_w,), jnp.int32),
            *[pltpu.VMEM((_CH, D), jnp.float32) for _ in range(_NBUF)],
            *[pltpu.SemaphoreType.DMA for _ in range(2 * _NBUF)],
        ],
    )
    def gather_kernel(idx_hbm, table_hbm, out_hbm, idx_v, *bufs_and_sems):
        bufs = bufs_and_sems[:_NBUF]
        gsem = bufs_and_sems[_NBUF : 2 * _NBUF]
        osem = bufs_and_sems[2 * _NBUF :]
        wid = lax.axis_index("s") * NC + lax.axis_index("c")
        base = wid * b_per_w
        pltpu.sync_copy(idx_hbm.at[pl.ds(base, b_per_w)], idx_v)

        def start_gather(c, b):
            pltpu.async_copy(
                table_hbm.at[idx_v.at[pl.ds(c * _CH, _CH)]], bufs[b], gsem[b]
            )

        def wait_gather(c, b):
            pltpu.make_async_copy(
                table_hbm.at[idx_v.at[pl.ds(c * _CH, _CH)]], bufs[b], gsem[b]
            ).wait()

        def start_out(c, b):
            pltpu.async_copy(
                bufs[b], out_hbm.at[pl.ds(base + c * _CH, _CH)], osem[b]
            )

        def wait_out(c, b):
            pltpu.make_async_copy(
                bufs[b], out_hbm.at[pl.ds(base + c * _CH, _CH)], osem[b]
            ).wait()

        start_gather(0, 0)
        start_gather(1, 1)
        for b in range(_NBUF):
            c = b
            if c >= 2:
                wait_out(c - 2, (b + 2) % _NBUF)
            start_gather(c + 2, (b + 2) % _NBUF)
            wait_gather(c, b)
            start_out(c, b)

        def ring(r, carry):
            for b in range(_NBUF):
                c = r * _NBUF + b
                wait_out(c - 2, (b + 2) % _NBUF)
                start_gather(c + 2, (b + 2) % _NBUF)
                wait_gather(c, b)
                start_out(c, b)
            return carry

        lax.fori_loop(1, n_rings - 1, ring, 0)

        for b in range(_NBUF):
            c = (n_rings - 1) * _NBUF + b
            if c + 2 < n_chunks:
                wait_out(c - 2, (b + 2) % _NBUF)
                start_gather(c + 2, (b + 2) % _NBUF)
            wait_gather(c, b)
            start_out(c, b)

        for c in (n_chunks - 2, n_chunks - 1):
            wait_out(c, c % _NBUF)

    return gather_kernel


def kernel(input_ids, wte):
    input_shape = input_ids.shape
    flat_ids = input_ids.reshape(-1).astype(jnp.int32)
    B = flat_ids.shape[0]
    D = wte.shape[1]
    out = _make_gather(B, D)(flat_ids, wte)
    return out.reshape(input_shape[0], input_shape[1], D)


# speedup vs baseline: 1.6657x; 1.0049x over previous
import functools

import jax
import jax.numpy as jnp
from jax import lax
from jax.experimental import pallas as pl
from jax.experimental.pallas import tpu as pltpu
from jax.experimental.pallas import tpu_sc as plsc

_NBUF = 3
_CH = 16


@functools.lru_cache(maxsize=None)
def _make_gather(B: int, D: int):
    info = plsc.get_sparse_core_info()
    NC, NS = info.num_cores, info.num_subcores
    NW = NC * NS
    b_per_w = B // NW
    n_chunks = b_per_w // _CH
    assert n_chunks % _NBUF == 1 and n_chunks >= 2 * _NBUF
    n_full_rings = (n_chunks - 1) // _NBUF
    mesh = plsc.VectorSubcoreMesh(core_axis_name="c", subcore_axis_name="s")

    @functools.partial(
        pl.kernel,
        mesh=mesh,
        out_type=jax.ShapeDtypeStruct((B, D), jnp.float32),
        scratch_types=[
            pltpu.VMEM((b_per_w,), jnp.int32),
            *[pltpu.VMEM((_CH, D), jnp.float32) for _ in range(_NBUF)],
            *[pltpu.SemaphoreType.DMA for _ in range(2 * _NBUF)],
        ],
    )
    def gather_kernel(idx_hbm, table_hbm, out_hbm, idx_v, *bufs_and_sems):
        bufs = bufs_and_sems[:_NBUF]
        gsem = bufs_and_sems[_NBUF : 2 * _NBUF]
        osem = bufs_and_sems[2 * _NBUF :]
        wid = lax.axis_index("s") * NC + lax.axis_index("c")
        base = wid * b_per_w
        pltpu.sync_copy(idx_hbm.at[pl.ds(base, b_per_w)], idx_v)

        def start_gather(c, b):
            pltpu.async_copy(
                table_hbm.at[idx_v.at[pl.ds(c * _CH, _CH)]], bufs[b], gsem[b]
            )

        def wait_gather(c, b):
            pltpu.make_async_copy(
                table_hbm.at[idx_v.at[pl.ds(c * _CH, _CH)]], bufs[b], gsem[b]
            ).wait()

        def start_out(c, b):
            pltpu.async_copy(
                bufs[b], out_hbm.at[pl.ds(base + c * _CH, _CH)], osem[b]
            )

        def wait_out(c, b):
            pltpu.make_async_copy(
                bufs[b], out_hbm.at[pl.ds(base + c * _CH, _CH)], osem[b]
            ).wait()

        start_gather(0, 0)
        start_gather(1, 1)

        start_gather(2, 2)
        wait_gather(0, 0)
        start_out(0, 0)
        for b in (1, 2):
            c = b
            wait_out(c - 1, (b + 2) % _NBUF)
            start_gather(c + 2, (b + 2) % _NBUF)
            wait_gather(c, b)
            start_out(c, b)

        def ring(r, carry):
            for b in range(_NBUF):
                c = r * _NBUF + b
                wait_out(c - 1, (b + 2) % _NBUF)
                start_gather(c + 2, (b + 2) % _NBUF)
                wait_gather(c, b)
                start_out(c, b)
            return carry

        lax.fori_loop(1, n_full_rings - 1, ring, 0)

        for c in range((n_full_rings - 1) * _NBUF, n_chunks):
            b = c % _NBUF
            if c + 2 < n_chunks:
                wait_out(c - 1, (b + 2) % _NBUF)
                start_gather(c + 2, (b + 2) % _NBUF)
            wait_gather(c, b)
            start_out(c, b)

        for c in range(n_chunks - _NBUF, n_chunks):
            wait_out(c, c % _NBUF)

    return gather_kernel


def kernel(input_ids, wte):
    input_shape = input_ids.shape
    flat_ids = input_ids.reshape(-1).astype(jnp.int32)
    B = flat_ids.shape[0]
    D = wte.shape[1]
    out = _make_gather(B, D)(flat_ids, wte)
    return out.reshape(input_shape[0], input_shape[1], D)
